# baseline (speedup 1.0000x reference)
import functools
import math

import jax
import jax.numpy as jnp
from jax import lax
from jax.experimental import pallas as pl
from jax.experimental.pallas import tpu as pltpu
from jax.experimental.pallas import tpu_sc as plsc

VOCAB = 250027
D = 1024
B = 32
S = 1024
L = 16
NV = D // L
NC = 2
NS = 16
NW = NC * NS
CHUNKS = S // NW
SCALE = math.sqrt(float(D))
EPS = 1e-5


def _body(tok_hbm, weight_hbm, pos_hbm, gamma_hbm, beta_hbm, out_hbm,
          tok_v, pos_v, gam_v, bet_v, buf0, buf1, gs0, gs1, ss0, ss1):
    wid = lax.axis_index("s") * NC + lax.axis_index("c")
    s0 = wid * CHUNKS
    row0 = s0 * B

    pltpu.sync_copy(tok_hbm.at[pl.ds(s0, CHUNKS)], tok_v)
    pltpu.sync_copy(pos_hbm.at[pl.ds(s0, CHUNKS)], pos_v)
    pltpu.sync_copy(gamma_hbm, gam_v)
    pltpu.sync_copy(beta_hbm, bet_v)

    def lane_sum(x):
        dnums = lax.GatherDimensionNumbers(
            offset_dims=(), collapsed_slice_dims=(0,), start_index_map=(0,))
        for sh in (8, 4, 2, 1):
            idx = (lax.iota(jnp.int32, L) + sh) % L
            x = x + lax.gather(x, idx[:, None], dnums, slice_sizes=(1,),
                               mode=lax.GatherScatterMode.PROMISE_IN_BOUNDS)
        return x

    bufs = (buf0, buf1)
    gsems = (gs0, gs1)
    ssems = (ss0, ss1)

    pltpu.async_copy(weight_hbm.at[tok_v.at[0]], buf0, gs0)
    pltpu.async_copy(weight_hbm.at[tok_v.at[1]], buf1, gs1)

    def compute_chunk(c, buf):
        zeros = jnp.zeros((L,), jnp.float32)

        def row_body(r, _):
            def p1(j, carry):
                acc, acc2 = carry
                sl = pl.ds(j * L, L)
                x = buf[r, sl] * SCALE + pos_v[c, sl]
                buf[r, sl] = x
                return acc + x, acc2 + x * x

            acc, acc2 = lax.fori_loop(0, NV, p1, (zeros, zeros), unroll=4)
            mean = lane_sum(acc) * (1.0 / D)
            var = lane_sum(acc2) * (1.0 / D) - mean * mean
            a = var + EPS
            seed = lax.bitcast_convert_type(
                jnp.int32(0x5F3759DF)
                - (lax.bitcast_convert_type(a, jnp.int32) >> 1),
                jnp.float32)
            y = seed
            for _ in range(3):
                y = y * (1.5 - 0.5 * a * y * y)

            def p2(j, _):
                sl = pl.ds(j * L, L)
                x = buf[r, sl]
                buf[r, sl] = (x - mean) * y * gam_v[sl] + bet_v[sl]
                return 0

            lax.fori_loop(0, NV, p2, 0, unroll=4)
            return 0

        lax.fori_loop(0, B, row_body, 0)

    def pair_body(i, _):
        for b in range(2):
            c = 2 * i + b
            buf, gs, ss = bufs[b], gsems[b], ssems[b]
            pltpu.make_async_copy(weight_hbm.at[tok_v.at[c]], buf, gs).wait()
            compute_chunk(c, buf)
            dst = out_hbm.at[pl.ds(row0 + c * B, B)]
            pltpu.async_copy(buf, dst, ss)

            @pl.when(c + 2 < CHUNKS)
            def _():
                pltpu.make_async_copy(buf, dst, ss).wait()
                pltpu.async_copy(weight_hbm.at[tok_v.at[c + 2]], buf, gs)

        return 0

    lax.fori_loop(0, CHUNKS // 2, pair_body, 0)

    for b in range(2):
        c = CHUNKS - 2 + b
        pltpu.make_async_copy(bufs[b], out_hbm.at[pl.ds(row0 + c * B, B)],
                              ssems[b]).wait()


@jax.jit
def _sc_embed_ln(tokens_t, weight, pos_weight, ln_gamma, ln_beta):
    mesh = plsc.VectorSubcoreMesh(core_axis_name="c", subcore_axis_name="s")
    return pl.kernel(
        _body,
        mesh=mesh,
        out_type=jax.ShapeDtypeStruct((S * B, D), jnp.float32),
        scratch_types=[
            pltpu.VMEM((CHUNKS, B), jnp.int32),
            pltpu.VMEM((CHUNKS, D), jnp.float32),
            pltpu.VMEM((D,), jnp.float32),
            pltpu.VMEM((D,), jnp.float32),
            pltpu.VMEM((B, D), jnp.float32),
            pltpu.VMEM((B, D), jnp.float32),
            pltpu.SemaphoreType.DMA,
            pltpu.SemaphoreType.DMA,
            pltpu.SemaphoreType.DMA,
            pltpu.SemaphoreType.DMA,
        ],
    )(tokens_t, weight, pos_weight, ln_gamma, ln_beta)


def kernel(tokens, weight, pos_weight, ln_gamma, ln_beta):
    tokens_t = jnp.transpose(tokens)
    out = _sc_embed_ln(tokens_t, weight, pos_weight, ln_gamma, ln_beta)
    return out.reshape(S, B, D)


# speedup vs baseline: 3.8842x; 3.8842x over previous
import functools
import math

import jax
import jax.numpy as jnp
from jax import lax
from jax.experimental import pallas as pl
from jax.experimental.pallas import tpu as pltpu
from jax.experimental.pallas import tpu_sc as plsc

VOCAB = 250027
D = 1024
B = 32
S = 1024
NC = 2
NS = 16
NW = NC * NS
CHUNKS = S // NW
SCALE = math.sqrt(float(D))
EPS = 1e-5
SB = 8


def _sc_gather_body(tok_hbm, weight_hbm, out_hbm,
                    tok_v, buf0, buf1, gs0, gs1, ss0, ss1):
    wid = lax.axis_index("s") * NC + lax.axis_index("c")
    s0 = wid * CHUNKS
    row0 = s0 * B

    pltpu.sync_copy(tok_hbm.at[pl.ds(s0, CHUNKS)], tok_v)

    bufs = (buf0, buf1)
    gsems = (gs0, gs1)
    ssems = (ss0, ss1)

    pltpu.async_copy(weight_hbm.at[tok_v.at[0]], buf0, gs0)
    pltpu.async_copy(weight_hbm.at[tok_v.at[1]], buf1, gs1)

    def pair_body(i, _):
        for b in range(2):
            c = 2 * i + b
            buf, gs, ss = bufs[b], gsems[b], ssems[b]
            pltpu.make_async_copy(weight_hbm.at[tok_v.at[c]], buf, gs).wait()
            dst = out_hbm.at[pl.ds(row0 + c * B, B)]
            pltpu.async_copy(buf, dst, ss)

            @pl.when(c + 2 < CHUNKS)
            def _():
                pltpu.make_async_copy(buf, dst, ss).wait()
                pltpu.async_copy(weight_hbm.at[tok_v.at[c + 2]], buf, gs)

        return 0

    lax.fori_loop(0, CHUNKS // 2, pair_body, 0)

    for b in range(2):
        c = CHUNKS - 2 + b
        pltpu.make_async_copy(bufs[b], out_hbm.at[pl.ds(row0 + c * B, B)],
                              ssems[b]).wait()


def _sc_gather(tokens_t, weight):
    mesh = plsc.VectorSubcoreMesh(core_axis_name="c", subcore_axis_name="s")
    return pl.kernel(
        _sc_gather_body,
        mesh=mesh,
        out_type=jax.ShapeDtypeStruct((S * B, D), jnp.float32),
        scratch_types=[
            pltpu.VMEM((CHUNKS, B), jnp.int32),
            pltpu.VMEM((B, D), jnp.float32),
            pltpu.VMEM((B, D), jnp.float32),
            pltpu.SemaphoreType.DMA,
            pltpu.SemaphoreType.DMA,
            pltpu.SemaphoreType.DMA,
            pltpu.SemaphoreType.DMA,
        ],
    )(tokens_t, weight)


def _tc_ln_body(pos_ref, gam_ref, bet_ref, g_ref, o_ref):
    x = g_ref[...].reshape(SB, B, D) * SCALE + pos_ref[...][:, None, :]
    mean = jnp.mean(x, axis=-1, keepdims=True)
    xc = x - mean
    var = jnp.mean(xc * xc, axis=-1, keepdims=True)
    y = xc * lax.rsqrt(var + EPS) * gam_ref[...][None, :, :] + bet_ref[...]
    o_ref[...] = y.reshape(SB * B, D)


def _tc_ln(gath, pos_weight, gamma2d, beta2d):
    grid = (S // SB,)
    return pl.pallas_call(
        _tc_ln_body,
        grid=grid,
        in_specs=[
            pl.BlockSpec((SB, D), lambda i: (i, 0)),
            pl.BlockSpec((1, D), lambda i: (0, 0)),
            pl.BlockSpec((1, D), lambda i: (0, 0)),
            pl.BlockSpec((SB * B, D), lambda i: (i, 0)),
        ],
        out_specs=pl.BlockSpec((SB * B, D), lambda i: (i, 0)),
        out_shape=jax.ShapeDtypeStruct((S * B, D), jnp.float32),
        compiler_params=pltpu.CompilerParams(
            dimension_semantics=("arbitrary",)),
    )(pos_weight, gamma2d, beta2d, gath)


@jax.jit
def _shard_head_tail(tokens, weight, pos_weight, ln_gamma, ln_beta):
    tokens_t = jnp.transpose(tokens)
    gath = _sc_gather(tokens_t, weight)
    out = _tc_ln(gath, pos_weight,
                 ln_gamma.reshape(1, D), ln_beta.reshape(1, D))
    return out.reshape(S, B, D)


def kernel(tokens, weight, pos_weight, ln_gamma, ln_beta):
    return _shard_head_tail(tokens, weight, pos_weight, ln_gamma, ln_beta)


# speedup vs baseline: 4.5245x; 1.1649x over previous
import functools
import math

import jax
import jax.numpy as jnp
from jax import lax
from jax.experimental import pallas as pl
from jax.experimental.pallas import tpu as pltpu
from jax.experimental.pallas import tpu_sc as plsc

VOCAB = 250027
D = 1024
B = 32
S = 1024
NC = 2
NS = 16
NW = NC * NS
CHUNKS = S // NW
SCALE = math.sqrt(float(D))
EPS = 1e-5
SB = 16


def _sc_gather_body(tok_hbm, weight_hbm, out_hbm,
                    tok_v, buf0, buf1, gs0, gs1, ss0, ss1):
    wid = lax.axis_index("s") * NC + lax.axis_index("c")
    s0 = wid * CHUNKS
    row0 = s0 * B

    pltpu.sync_copy(tok_hbm.at[pl.ds(s0, CHUNKS)], tok_v)

    bufs = (buf0, buf1)
    gsems = (gs0, gs1)
    ssems = (ss0, ss1)

    pltpu.async_copy(weight_hbm.at[tok_v.at[0]], buf0, gs0)
    pltpu.async_copy(weight_hbm.at[tok_v.at[1]], buf1, gs1)

    def pair_body(i, _):
        for b in range(2):
            c = 2 * i + b
            buf, gs, ss = bufs[b], gsems[b], ssems[b]
            pltpu.make_async_copy(weight_hbm.at[tok_v.at[c]], buf, gs).wait()
            dst = out_hbm.at[pl.ds(row0 + c * B, B)]
            pltpu.async_copy(buf, dst, ss)

            @pl.when(c + 2 < CHUNKS)
            def _():
                pltpu.make_async_copy(buf, dst, ss).wait()
                pltpu.async_copy(weight_hbm.at[tok_v.at[c + 2]], buf, gs)

        return 0

    lax.fori_loop(0, CHUNKS // 2, pair_body, 0)

    for b in range(2):
        c = CHUNKS - 2 + b
        pltpu.make_async_copy(bufs[b], out_hbm.at[pl.ds(row0 + c * B, B)],
                              ssems[b]).wait()


def _sc_gather(tokens_t, weight):
    mesh = plsc.VectorSubcoreMesh(core_axis_name="c", subcore_axis_name="s")
    return pl.kernel(
        _sc_gather_body,
        mesh=mesh,
        out_type=jax.ShapeDtypeStruct((S * B, D), jnp.float32),
        scratch_types=[
            pltpu.VMEM((CHUNKS, B), jnp.int32),
            pltpu.VMEM((B, D), jnp.float32),
            pltpu.VMEM((B, D), jnp.float32),
            pltpu.SemaphoreType.DMA,
            pltpu.SemaphoreType.DMA,
            pltpu.SemaphoreType.DMA,
            pltpu.SemaphoreType.DMA,
        ],
    )(tokens_t, weight)


def _tc_ln_body(pos_ref, gam_ref, bet_ref, g_ref, o_ref):
    x = g_ref[...].reshape(SB, B, D) * SCALE + pos_ref[...][:, None, :]
    mean = jnp.mean(x, axis=-1, keepdims=True)
    xc = x - mean
    var = jnp.mean(xc * xc, axis=-1, keepdims=True)
    y = xc * lax.rsqrt(var + EPS) * gam_ref[...][None, :, :] + bet_ref[...]
    o_ref[...] = y.reshape(SB * B, D)


def _tc_ln(gath, pos_weight, gamma2d, beta2d):
    grid = (S // SB,)
    return pl.pallas_call(
        _tc_ln_body,
        grid=grid,
        in_specs=[
            pl.BlockSpec((SB, D), lambda i: (i, 0)),
            pl.BlockSpec((1, D), lambda i: (0, 0)),
            pl.BlockSpec((1, D), lambda i: (0, 0)),
            pl.BlockSpec((SB * B, D), lambda i: (i, 0)),
        ],
        out_specs=pl.BlockSpec((SB * B, D), lambda i: (i, 0)),
        out_shape=jax.ShapeDtypeStruct((S * B, D), jnp.float32),
        compiler_params=pltpu.CompilerParams(
            dimension_semantics=("arbitrary",)),
    )(pos_weight, gamma2d, beta2d, gath)


@jax.jit
def _shard_head_tail(tokens, weight, pos_weight, ln_gamma, ln_beta):
    tokens_t = jnp.transpose(tokens)
    gath = _sc_gather(tokens_t, weight)
    out = _tc_ln(gath, pos_weight,
                 ln_gamma.reshape(1, D), ln_beta.reshape(1, D))
    return out.reshape(S, B, D)


def kernel(tokens, weight, pos_weight, ln_gamma, ln_beta):
    return _shard_head_tail(tokens, weight, pos_weight, ln_gamma, ln_beta)


# speedup vs baseline: 4.9114x; 1.0855x over previous
import functools
import math

import jax
import jax.numpy as jnp
from jax import lax
from jax.experimental import pallas as pl
from jax.experimental.pallas import tpu as pltpu
from jax.experimental.pallas import tpu_sc as plsc

VOCAB = 250027
D = 1024
B = 32
S = 1024
NC = 2
NS = 16
NW = NC * NS
CHUNKS = S // NW
SCALE = math.sqrt(float(D))
EPS = 1e-5
SB = 32


def _sc_gather_body(tok_hbm, weight_hbm, out_hbm,
                    tok_v, buf0, buf1, gs0, gs1, ss0, ss1):
    wid = lax.axis_index("s") * NC + lax.axis_index("c")
    s0 = wid * CHUNKS
    row0 = s0 * B

    pltpu.sync_copy(tok_hbm.at[pl.ds(s0, CHUNKS)], tok_v)

    bufs = (buf0, buf1)
    gsems = (gs0, gs1)
    ssems = (ss0, ss1)

    pltpu.async_copy(weight_hbm.at[tok_v.at[0]], buf0, gs0)
    pltpu.async_copy(weight_hbm.at[tok_v.at[1]], buf1, gs1)

    def pair_body(i, _):
        for b in range(2):
            c = 2 * i + b
            buf, gs, ss = bufs[b], gsems[b], ssems[b]
            pltpu.make_async_copy(weight_hbm.at[tok_v.at[c]], buf, gs).wait()
            dst = out_hbm.at[pl.ds(row0 + c * B, B)]
            pltpu.async_copy(buf, dst, ss)

            @pl.when(c + 2 < CHUNKS)
            def _():
                pltpu.make_async_copy(buf, dst, ss).wait()
                pltpu.async_copy(weight_hbm.at[tok_v.at[c + 2]], buf, gs)

        return 0

    lax.fori_loop(0, CHUNKS // 2, pair_body, 0)

    for b in range(2):
        c = CHUNKS - 2 + b
        pltpu.make_async_copy(bufs[b], out_hbm.at[pl.ds(row0 + c * B, B)],
                              ssems[b]).wait()


def _sc_gather(tokens_t, weight):
    mesh = plsc.VectorSubcoreMesh(core_axis_name="c", subcore_axis_name="s")
    return pl.kernel(
        _sc_gather_body,
        mesh=mesh,
        out_type=jax.ShapeDtypeStruct((S * B, D), jnp.float32),
        scratch_types=[
            pltpu.VMEM((CHUNKS, B), jnp.int32),
            pltpu.VMEM((B, D), jnp.float32),
            pltpu.VMEM((B, D), jnp.float32),
            pltpu.SemaphoreType.DMA,
            pltpu.SemaphoreType.DMA,
            pltpu.SemaphoreType.DMA,
            pltpu.SemaphoreType.DMA,
        ],
    )(tokens_t, weight)


def _tc_ln_body(pos_ref, gam_ref, bet_ref, g_ref, o_ref):
    x = g_ref[...].reshape(SB, B, D) * SCALE + pos_ref[...][:, None, :]
    mean = jnp.mean(x, axis=-1, keepdims=True)
    xc = x - mean
    var = jnp.mean(xc * xc, axis=-1, keepdims=True)
    y = xc * lax.rsqrt(var + EPS) * gam_ref[...][None, :, :] + bet_ref[...]
    o_ref[...] = y.reshape(SB * B, D)


def _tc_ln(gath, pos_weight, gamma2d, beta2d):
    grid = (S // SB,)
    return pl.pallas_call(
        _tc_ln_body,
        grid=grid,
        in_specs=[
            pl.BlockSpec((SB, D), lambda i: (i, 0)),
            pl.BlockSpec((1, D), lambda i: (0, 0)),
            pl.BlockSpec((1, D), lambda i: (0, 0)),
            pl.BlockSpec((SB * B, D), lambda i: (i, 0)),
        ],
        out_specs=pl.BlockSpec((SB * B, D), lambda i: (i, 0)),
        out_shape=jax.ShapeDtypeStruct((S * B, D), jnp.float32),
        compiler_params=pltpu.CompilerParams(
            dimension_semantics=("arbitrary",)),
    )(pos_weight, gamma2d, beta2d, gath)


@jax.jit
def _shard_head_tail(tokens, weight, pos_weight, ln_gamma, ln_beta):
    tokens_t = jnp.transpose(tokens)
    gath = _sc_gather(tokens_t, weight)
    out = _tc_ln(gath, pos_weight,
                 ln_gamma.reshape(1, D), ln_beta.reshape(1, D))
    return out.reshape(S, B, D)


def kernel(tokens, weight, pos_weight, ln_gamma, ln_beta):
    return _shard_head_tail(tokens, weight, pos_weight, ln_gamma, ln_beta)


# speedup vs baseline: 4.9733x; 1.0126x over previous
import functools
import math

import jax
import jax.numpy as jnp
from jax import lax
from jax.experimental import pallas as pl
from jax.experimental.pallas import tpu as pltpu
from jax.experimental.pallas import tpu_sc as plsc

VOCAB = 250027
D = 1024
B = 32
S = 1024
NC = 2
NS = 16
NW = NC * NS
CHUNKS = S // NW
SCALE = math.sqrt(float(D))
EPS = 1e-5
SB = 64


def _sc_gather_body(tok_hbm, weight_hbm, out_hbm,
                    tok_v, buf0, buf1, gs0, gs1, ss0, ss1):
    wid = lax.axis_index("s") * NC + lax.axis_index("c")
    s0 = wid * CHUNKS
    row0 = s0 * B

    pltpu.sync_copy(tok_hbm.at[pl.ds(s0, CHUNKS)], tok_v)

    bufs = (buf0, buf1)
    gsems = (gs0, gs1)
    ssems = (ss0, ss1)

    pltpu.async_copy(weight_hbm.at[tok_v.at[0]], buf0, gs0)
    pltpu.async_copy(weight_hbm.at[tok_v.at[1]], buf1, gs1)

    def pair_body(i, _):
        for b in range(2):
            c = 2 * i + b
            buf, gs, ss = bufs[b], gsems[b], ssems[b]
            pltpu.make_async_copy(weight_hbm.at[tok_v.at[c]], buf, gs).wait()
            dst = out_hbm.at[pl.ds(row0 + c * B, B)]
            pltpu.async_copy(buf, dst, ss)

            @pl.when(c + 2 < CHUNKS)
            def _():
                pltpu.make_async_copy(buf, dst, ss).wait()
                pltpu.async_copy(weight_hbm.at[tok_v.at[c + 2]], buf, gs)

        return 0

    lax.fori_loop(0, CHUNKS // 2, pair_body, 0)

    for b in range(2):
        c = CHUNKS - 2 + b
        pltpu.make_async_copy(bufs[b], out_hbm.at[pl.ds(row0 + c * B, B)],
                              ssems[b]).wait()


def _sc_gather(tokens_t, weight):
    mesh = plsc.VectorSubcoreMesh(core_axis_name="c", subcore_axis_name="s")
    return pl.kernel(
        _sc_gather_body,
        mesh=mesh,
        out_type=jax.ShapeDtypeStruct((S * B, D), jnp.float32),
        scratch_types=[
            pltpu.VMEM((CHUNKS, B), jnp.int32),
            pltpu.VMEM((B, D), jnp.float32),
            pltpu.VMEM((B, D), jnp.float32),
            pltpu.SemaphoreType.DMA,
            pltpu.SemaphoreType.DMA,
            pltpu.SemaphoreType.DMA,
            pltpu.SemaphoreType.DMA,
        ],
    )(tokens_t, weight)


def _tc_ln_body(pos_ref, gam_ref, bet_ref, g_ref, o_ref):
    x = g_ref[...].reshape(SB, B, D) * SCALE + pos_ref[...][:, None, :]
    mean = jnp.mean(x, axis=-1, keepdims=True)
    xc = x - mean
    var = jnp.mean(xc * xc, axis=-1, keepdims=True)
    y = xc * lax.rsqrt(var + EPS) * gam_ref[...][None, :, :] + bet_ref[...]
    o_ref[...] = y.reshape(SB * B, D)


def _tc_ln(gath, pos_weight, gamma2d, beta2d):
    grid = (S // SB,)
    return pl.pallas_call(
        _tc_ln_body,
        grid=grid,
        in_specs=[
            pl.BlockSpec((SB, D), lambda i: (i, 0)),
            pl.BlockSpec((1, D), lambda i: (0, 0)),
            pl.BlockSpec((1, D), lambda i: (0, 0)),
            pl.BlockSpec((SB * B, D), lambda i: (i, 0)),
        ],
        out_specs=pl.BlockSpec((SB * B, D), lambda i: (i, 0)),
        out_shape=jax.ShapeDtypeStruct((S * B, D), jnp.float32),
        compiler_params=pltpu.CompilerParams(
            dimension_semantics=("arbitrary",)),
    )(pos_weight, gamma2d, beta2d, gath)


@jax.jit
def _shard_head_tail(tokens, weight, pos_weight, ln_gamma, ln_beta):
    tokens_t = jnp.transpose(tokens)
    gath = _sc_gather(tokens_t, weight)
    out = _tc_ln(gath, pos_weight,
                 ln_gamma.reshape(1, D), ln_beta.reshape(1, D))
    return out.reshape(S, B, D)


def kernel(tokens, weight, pos_weight, ln_gamma, ln_beta):
    return _shard_head_tail(tokens, weight, pos_weight, ln_gamma, ln_beta)
